# baseline (speedup 1.0000x reference)
import functools

import jax
import jax.numpy as jnp
from jax import lax
from jax.experimental import pallas as pl
from jax.experimental.pallas import tpu as pltpu
from jax.experimental.pallas import tpu_sc as plsc

NUM_EMB = 103
EMB_DIM = 32
BATCH = 1024
SEQ = 50
MAX_PAD = 20

B_TOTAL = BATCH * SEQ * MAX_PAD
NUM_CORES = 2
NUM_SUBCORES = 16
NUM_WORKERS = NUM_CORES * NUM_SUBCORES
ROWS_PER_W = B_TOTAL // NUM_WORKERS
CHUNK = 800
NCHUNK = ROWS_PER_W // CHUNK
GROUPS = CHUNK // 16
CWORDS = CHUNK * EMB_DIM


@functools.partial(
    pl.kernel,
    out_type=jax.ShapeDtypeStruct((B_TOTAL * EMB_DIM,), jnp.float32),
    mesh=plsc.VectorSubcoreMesh(core_axis_name="c", subcore_axis_name="s"),
    scratch_types=[
        pltpu.VMEM((NUM_EMB * EMB_DIM,), jnp.float32),
        pltpu.VMEM((ROWS_PER_W,), jnp.int32),
        pltpu.VMEM((CWORDS,), jnp.float32),
        pltpu.VMEM((CWORDS,), jnp.float32),
        pltpu.SemaphoreType.DMA,
        pltpu.SemaphoreType.DMA,
    ],
    compiler_params=pltpu.CompilerParams(needs_layout_passes=False),
)
def _gather_rows(idx_hbm, table_hbm, out_hbm, table_v, idx_v, rows0, rows1,
                 so0, so1):
    wid = lax.axis_index("s") * NUM_CORES + lax.axis_index("c")
    base = wid * (ROWS_PER_W * EMB_DIM)
    rows = (rows0, rows1)
    so = (so0, so1)

    pltpu.sync_copy(table_hbm, table_v)
    pltpu.sync_copy(idx_hbm.at[pl.ds(wid * ROWS_PER_W, ROWS_PER_W)], idx_v)

    lane = lax.iota(jnp.int32, 16)
    lane32 = lane * EMB_DIM

    def fill_chunk(buf, g):
        @plsc.parallel_loop(0, GROUPS, 1, unroll=2)
        def group_body(u):
            vidx = idx_v[pl.ds(g * CHUNK + u * 16, 16)]
            vbase = vidx * EMB_DIM
            sbase = lane32 + u * (16 * EMB_DIM)
            for c in range(EMB_DIM):
                offv = (lane + c) & (EMB_DIM - 1)
                col = plsc.load_gather(table_v, [vbase + offv])
                plsc.store_scatter(buf, [sbase + offv], col)

    def flush(b, g):
        pltpu.async_copy(rows[b], out_hbm.at[pl.ds(base + g * CWORDS, CWORDS)],
                         so[b])

    def wait_flush(b, g):
        pltpu.make_async_copy(rows[b],
                              out_hbm.at[pl.ds(base + g * CWORDS, CWORDS)],
                              so[b]).wait()

    @pl.loop(0, NCHUNK, step=2)
    def chunk_pair(g):
        for b in range(2):
            @pl.when(g + b >= 2)
            def _():
                wait_flush(b, g + b - 2)

            fill_chunk(rows[b], g + b)
            flush(b, g + b)

    wait_flush(0, NCHUNK - 2)
    wait_flush(1, NCHUNK - 1)


def kernel(indices, table):
    B, S, P = indices.shape
    idx = indices.reshape(-1).astype(jnp.int32)
    table_flat = table.astype(jnp.float32).reshape(-1)
    out = _gather_rows(idx, table_flat)
    return out.reshape(B, S, P * table.shape[1])


# speedup vs baseline: 2.6415x; 2.6415x over previous
import functools

import jax
import jax.numpy as jnp
from jax import lax
from jax.experimental import pallas as pl
from jax.experimental.pallas import tpu as pltpu
from jax.experimental.pallas import tpu_sc as plsc

NUM_EMB = 103
EMB_DIM = 32
BATCH = 1024
SEQ = 50
MAX_PAD = 20
FEAT = MAX_PAD * EMB_DIM

NUM_CORES = 2
NUM_SUBCORES = 16
NUM_WORKERS = NUM_CORES * NUM_SUBCORES
BCOL = BATCH // NUM_WORKERS
BGROUPS = BCOL // 16
SLAB_ITERS = MAX_PAD * BGROUPS
IDX_PER_W = BCOL * SEQ * MAX_PAD


@functools.partial(
    pl.kernel,
    out_type=jax.ShapeDtypeStruct((SEQ, BATCH, FEAT), jnp.float32),
    mesh=plsc.VectorSubcoreMesh(core_axis_name="c", subcore_axis_name="s"),
    scratch_types=[
        pltpu.VMEM((NUM_EMB * EMB_DIM,), jnp.float32),
        pltpu.VMEM((IDX_PER_W,), jnp.int32),
        pltpu.VMEM((BCOL, FEAT), jnp.float32),
        pltpu.VMEM((BCOL, FEAT), jnp.float32),
        pltpu.SemaphoreType.DMA,
        pltpu.SemaphoreType.DMA,
    ],
    compiler_params=pltpu.CompilerParams(needs_layout_passes=False),
)
def _gather_rows(idx_hbm, table_hbm, out_hbm, table_v, idx_v, rows0, rows1,
                 so0, so1):
    wid = lax.axis_index("s") * NUM_CORES + lax.axis_index("c")
    b0 = wid * BCOL
    rows = (rows0, rows1)
    so = (so0, so1)

    pltpu.sync_copy(table_hbm, table_v)
    pltpu.sync_copy(idx_hbm.at[pl.ds(wid * IDX_PER_W, IDX_PER_W)], idx_v)

    lane = lax.iota(jnp.int32, 16)
    lane_sp = lane * (SEQ * MAX_PAD)

    def fill_slab(buf, s):
        @plsc.parallel_loop(0, SLAB_ITERS, 1, unroll=2)
        def slab_body(m):
            p = m >> 1
            bg = m & 1
            vidx = plsc.load_gather(
                idx_v, [lane_sp + (bg * 16 * SEQ * MAX_PAD + s * MAX_PAD + p)])
            vbase = vidx * EMB_DIM
            rowv = lane + bg * 16
            for c in range(EMB_DIM):
                offv = (lane + c) & (EMB_DIM - 1)
                col = plsc.load_gather(table_v, [vbase + offv])
                plsc.store_scatter(buf, [rowv, offv + p * EMB_DIM], col)

    def flush(b, s):
        pltpu.async_copy(rows[b], out_hbm.at[s, pl.ds(b0, BCOL)], so[b])

    def wait_flush(b, s):
        pltpu.make_async_copy(rows[b], out_hbm.at[s, pl.ds(b0, BCOL)],
                              so[b]).wait()

    @pl.loop(0, SEQ, step=2)
    def s_pair(s):
        for b in range(2):
            @pl.when(s + b >= 2)
            def _():
                wait_flush(b, s + b - 2)

            fill_slab(rows[b], s + b)
            flush(b, s + b)

    wait_flush(0, SEQ - 2)
    wait_flush(1, SEQ - 1)


def kernel(indices, table):
    B, S, P = indices.shape
    idx = indices.astype(jnp.int32).reshape(-1)
    table_flat = table.astype(jnp.float32).reshape(-1)
    out = _gather_rows(idx, table_flat)
    return jnp.transpose(out, (1, 0, 2))


# speedup vs baseline: 2.8349x; 1.0732x over previous
import functools

import jax
import jax.numpy as jnp
from jax import lax
from jax.experimental import pallas as pl
from jax.experimental.pallas import tpu as pltpu
from jax.experimental.pallas import tpu_sc as plsc

NUM_EMB = 103
EMB_DIM = 32
BATCH = 1024
SEQ = 50
MAX_PAD = 20
FEAT = MAX_PAD * EMB_DIM

NUM_CORES = 2
NUM_SUBCORES = 16
NUM_WORKERS = NUM_CORES * NUM_SUBCORES
BCOL = BATCH // NUM_WORKERS
SLAB_ITERS = MAX_PAD * (BCOL // 16)
STILE = 8
NSTILE = SEQ // STILE
STAIL = SEQ - NSTILE * STILE


@functools.partial(
    pl.kernel,
    out_type=jax.ShapeDtypeStruct((SEQ, BATCH, FEAT), jnp.float32),
    mesh=plsc.VectorSubcoreMesh(core_axis_name="c", subcore_axis_name="s"),
    scratch_types=[
        pltpu.VMEM((NUM_EMB * EMB_DIM,), jnp.float32),
        pltpu.VMEM((MAX_PAD, STILE, 128), jnp.int32),
        pltpu.VMEM((MAX_PAD, STILE, 128), jnp.int32),
        pltpu.VMEM((MAX_PAD, STAIL, 128), jnp.int32),
        pltpu.VMEM((BCOL, FEAT), jnp.float32),
        pltpu.VMEM((BCOL, FEAT), jnp.float32),
        pltpu.SemaphoreType.DMA,
        pltpu.SemaphoreType.DMA,
        pltpu.SemaphoreType.DMA,
        pltpu.SemaphoreType.DMA,
        pltpu.SemaphoreType.DMA,
    ],
    compiler_params=pltpu.CompilerParams(needs_layout_passes=False),
)
def _gather_rows(idx_hbm, table_hbm, out_hbm, table_v, iblk0, iblk1, itail,
                 rows0, rows1, si0, si1, sit, so0, so1):
    wid = lax.axis_index("s") * NUM_CORES + lax.axis_index("c")
    b0 = wid * BCOL
    bb = (wid // 4) * 128
    bq = (wid % 4) * BCOL
    iblk = (iblk0, iblk1)
    si = (si0, si1)
    rows = (rows0, rows1)
    so = (so0, so1)

    pltpu.sync_copy(table_hbm, table_v)

    def iblk_src(st):
        return idx_hbm.at[:, pl.ds(st * STILE, STILE), pl.ds(bb, 128)]

    pltpu.async_copy(iblk_src(0), iblk0, si0)
    pltpu.async_copy(iblk_src(1), iblk1, si1)
    pltpu.async_copy(idx_hbm.at[:, pl.ds(NSTILE * STILE, STAIL),
                                pl.ds(bb, 128)], itail, sit)

    lane = lax.iota(jnp.int32, 16)

    def fill_slab(buf, src, s_local):
        @plsc.parallel_loop(0, SLAB_ITERS, 1, unroll=2)
        def slab_body(m):
            p = m >> 1
            bg = m & 1
            vidx = plsc.load_gather(
                src, [jnp.full((16,), p, jnp.int32),
                      jnp.full((16,), s_local, jnp.int32),
                      lane + (bq + bg * 16)])
            vbase = vidx * EMB_DIM
            rowv = lane + bg * 16
            for c in range(EMB_DIM):
                offv = (lane + c) & (EMB_DIM - 1)
                col = plsc.load_gather(table_v, [vbase + offv])
                plsc.store_scatter(buf, [rowv, offv + p * EMB_DIM], col)

    def flush(b, s):
        pltpu.async_copy(rows[b], out_hbm.at[s, pl.ds(b0, BCOL)], so[b])

    def wait_flush(b, s):
        pltpu.make_async_copy(rows[b], out_hbm.at[s, pl.ds(b0, BCOL)],
                              so[b]).wait()

    @pl.loop(0, NSTILE)
    def s_tile(st):
        ib = lax.rem(st, 2)
        for b in range(2):
            @pl.when(ib == b)
            def _():
                pltpu.make_async_copy(iblk_src(st), iblk[b], si[b]).wait()

        @pl.loop(0, STILE, step=2)
        def s_pair(s_local):
            for b2 in range(2):
                s = st * STILE + s_local + b2

                @pl.when(s >= 2)
                def _():
                    wait_flush(b2, s - 2)

                for b in range(2):
                    @pl.when(ib == b)
                    def _():
                        fill_slab(rows[b2], iblk[b], s_local + b2)

                flush(b2, s)

        @pl.when(st + 2 < NSTILE)
        def _():
            for b in range(2):
                @pl.when(ib == b)
                def _():
                    pltpu.async_copy(iblk_src(st + 2), iblk[b], si[b])

    pltpu.make_async_copy(idx_hbm.at[:, pl.ds(NSTILE * STILE, STAIL),
                                     pl.ds(bb, 128)], itail, sit).wait()
    for t in range(STAIL):
        s = NSTILE * STILE + t
        wait_flush(t % 2, s - 2)
        fill_slab(rows[t % 2], itail, t)
        flush(t % 2, s)

    wait_flush(0, SEQ - 2)
    wait_flush(1, SEQ - 1)


def kernel(indices, table):
    B, S, P = indices.shape
    idx_t = jnp.transpose(indices.astype(jnp.int32), (2, 1, 0))
    table_flat = table.astype(jnp.float32).reshape(-1)
    out = _gather_rows(idx_t, table_flat)
    return jnp.transpose(out, (1, 0, 2))


# speedup vs baseline: 3.4399x; 1.2134x over previous
import functools

import jax
import jax.numpy as jnp
from jax import lax
from jax.experimental import pallas as pl
from jax.experimental.pallas import tpu as pltpu
from jax.experimental.pallas import tpu_sc as plsc

NUM_EMB = 103
EMB_DIM = 32
BATCH = 1024
SEQ = 50
MAX_PAD = 20
FEAT = MAX_PAD * EMB_DIM

NUM_CORES = 2
NUM_SUBCORES = 16
NUM_WORKERS = NUM_CORES * NUM_SUBCORES
BCOL = BATCH // NUM_WORKERS
SLAB_ITERS = MAX_PAD * (BCOL // 16)
STILE = 8
NSTILE = 6
STAIL = SEQ - NSTILE * STILE


@functools.partial(
    pl.kernel,
    out_type=jax.ShapeDtypeStruct((SEQ, BATCH, FEAT), jnp.float32),
    mesh=plsc.VectorSubcoreMesh(core_axis_name="c", subcore_axis_name="s"),
    scratch_types=[
        pltpu.VMEM((NUM_EMB * EMB_DIM,), jnp.float32),
        pltpu.VMEM((2, MAX_PAD, STILE, 128), jnp.int32),
        pltpu.VMEM((2, BCOL, FEAT), jnp.float32),
        pltpu.SemaphoreType.DMA,
        pltpu.SemaphoreType.DMA,
        pltpu.SemaphoreType.DMA,
        pltpu.SemaphoreType.DMA,
    ],
    compiler_params=pltpu.CompilerParams(needs_layout_passes=False),
)
def _gather_rows(idx_hbm, table_hbm, out_hbm, table_v, iring, rows,
                 si0, si1, so0, so1):
    wid = lax.axis_index("s") * NUM_CORES + lax.axis_index("c")
    b0 = wid * BCOL
    bb = (wid // 4) * 128
    bq = (wid % 4) * BCOL
    si = (si0, si1)
    so = (so0, so1)

    pltpu.sync_copy(table_hbm, table_v)

    def iblk_copy(bk, slot):
        return pltpu.make_async_copy(
            idx_hbm.at[:, pl.ds(bk * STILE, STILE), pl.ds(bb, 128)],
            iring.at[slot], si[slot])

    def itail_copy(slot):
        return pltpu.make_async_copy(
            idx_hbm.at[:, pl.ds(NSTILE * STILE, STAIL), pl.ds(bb, 128)],
            iring.at[slot].at[:, pl.ds(0, STAIL)], si[slot])

    iblk_copy(0, 0).start()
    iblk_copy(1, 1).start()

    lane = lax.iota(jnp.int32, 16)

    def fill_slab(slot, b2, s_local):
        @plsc.parallel_loop(0, SLAB_ITERS, 1, unroll=2)
        def slab_body(m):
            p = m >> 1
            bg = m & 1
            vidx = plsc.load_gather(
                iring, [jnp.full((16,), slot, jnp.int32),
                        jnp.full((16,), p, jnp.int32),
                        jnp.full((16,), s_local, jnp.int32),
                        lane + (bq + bg * 16)])
            vbase = vidx * EMB_DIM
            rowv = lane + bg * 16
            b2v = jnp.full((16,), b2, jnp.int32)
            for c in range(EMB_DIM):
                offv = (lane + c) & (EMB_DIM - 1)
                col = plsc.load_gather(table_v, [vbase + offv])
                plsc.store_scatter(rows, [b2v, rowv, offv + p * EMB_DIM], col)

    def flush_copy(b2s, s):
        return pltpu.make_async_copy(rows.at[b2s],
                                     out_hbm.at[s, pl.ds(b0, BCOL)], so[b2s])

    @pl.loop(0, SEQ)
    def s_loop(s):
        bk = s >> 3
        slot = lax.rem(bk, 2)
        s_local = lax.rem(s, STILE)
        b2 = lax.rem(s, 2)

        @pl.when(s_local == 0)
        def _():
            for sl in range(2):
                @pl.when(slot == sl)
                def _():
                    @pl.when(bk < NSTILE)
                    def _():
                        iblk_copy(bk, sl).wait()

                    @pl.when(bk == NSTILE)
                    def _():
                        itail_copy(sl).wait()

        @pl.when(s >= 2)
        def _():
            for b in range(2):
                @pl.when(b2 == b)
                def _():
                    flush_copy(b, s - 2).wait()

        fill_slab(slot, b2, s_local)

        for b in range(2):
            @pl.when(b2 == b)
            def _():
                flush_copy(b, s).start()

        @pl.when(s_local == STILE - 1)
        def _():
            for sl in range(2):
                @pl.when(slot == sl)
                def _():
                    @pl.when(bk + 2 < NSTILE)
                    def _():
                        iblk_copy(bk + 2, sl).start()

                    @pl.when(bk + 2 == NSTILE)
                    def _():
                        itail_copy(sl).start()

    flush_copy(0, SEQ - 2).wait()
    flush_copy(1, SEQ - 1).wait()


def kernel(indices, table):
    B, S, P = indices.shape
    idx_t = jnp.transpose(indices.astype(jnp.int32), (2, 1, 0))
    table_flat = table.astype(jnp.float32).reshape(-1)
    out = _gather_rows(idx_t, table_flat)
    return jnp.transpose(out, (1, 0, 2))


# speedup vs baseline: 3.4436x; 1.0011x over previous
import functools

import jax
import jax.numpy as jnp
from jax import lax
from jax.experimental import pallas as pl
from jax.experimental.pallas import tpu as pltpu
from jax.experimental.pallas import tpu_sc as plsc

NUM_EMB = 103
EMB_DIM = 32
BATCH = 1024
SEQ = 50
MAX_PAD = 20
FEAT = MAX_PAD * EMB_DIM

NUM_CORES = 2
NUM_SUBCORES = 16
NUM_WORKERS = NUM_CORES * NUM_SUBCORES
BCOL = BATCH // NUM_WORKERS
SLAB_ITERS = MAX_PAD * (BCOL // 16)
STILE = 8
NSTILE = 6
STAIL = SEQ - NSTILE * STILE


@functools.partial(
    pl.kernel,
    out_type=jax.ShapeDtypeStruct((SEQ, BATCH, FEAT), jnp.float32),
    mesh=plsc.VectorSubcoreMesh(core_axis_name="c", subcore_axis_name="s"),
    scratch_types=[
        pltpu.VMEM((NUM_EMB * EMB_DIM,), jnp.float32),
        pltpu.VMEM((2, MAX_PAD, STILE, 128), jnp.int32),
        pltpu.VMEM((2 * BCOL, FEAT), jnp.float32),
        pltpu.SemaphoreType.DMA,
        pltpu.SemaphoreType.DMA,
        pltpu.SemaphoreType.DMA,
        pltpu.SemaphoreType.DMA,
    ],
    compiler_params=pltpu.CompilerParams(needs_layout_passes=False),
)
def _gather_rows(idx_hbm, table_hbm, out_hbm, table_v, iring, rows,
                 si0, si1, so0, so1):
    wid = lax.axis_index("s") * NUM_CORES + lax.axis_index("c")
    b0 = wid * BCOL
    bb = (wid // 4) * 128
    bq = (wid % 4) * BCOL
    si = (si0, si1)
    so = (so0, so1)

    pltpu.sync_copy(table_hbm, table_v)

    def iblk_copy(bk, slot):
        return pltpu.make_async_copy(
            idx_hbm.at[:, pl.ds(bk * STILE, STILE), pl.ds(bb, 128)],
            iring.at[slot], si[slot])

    def itail_copy(slot):
        return pltpu.make_async_copy(
            idx_hbm.at[:, pl.ds(NSTILE * STILE, STAIL), pl.ds(bb, 128)],
            iring.at[slot].at[:, pl.ds(0, STAIL)], si[slot])

    iblk_copy(0, 0).start()
    iblk_copy(1, 1).start()

    lane = lax.iota(jnp.int32, 16)
    ones_mask = lane >= 0
    offs = [(lane + c) & (EMB_DIM - 1) for c in range(EMB_DIM)]

    def fill_slab(slot, b2, s_local):
        @plsc.parallel_loop(0, SLAB_ITERS, 1, unroll=2)
        def slab_body(m):
            p = m >> 1
            bg = m & 1
            vidx = plsc.load_gather(
                iring, [jnp.full((16,), slot, jnp.int32),
                        jnp.full((16,), p, jnp.int32),
                        jnp.full((16,), s_local, jnp.int32),
                        lane + (bq + bg * 16)],
                mask=ones_mask)
            vbase = vidx * EMB_DIM
            rowv = lane + (b2 * BCOL + bg * 16)
            p32 = p * EMB_DIM
            for c in range(EMB_DIM):
                col = plsc.load_gather(table_v, [vbase + offs[c]],
                                       mask=ones_mask)
                plsc.store_scatter(rows, [rowv, offs[c] + p32], col,
                                   mask=ones_mask)

    def flush_copy(b2s, s):
        return pltpu.make_async_copy(rows.at[pl.ds(b2s * BCOL, BCOL)],
                                     out_hbm.at[s, pl.ds(b0, BCOL)], so[b2s])

    @pl.loop(0, SEQ)
    def s_loop(s):
        bk = s >> 3
        slot = lax.rem(bk, 2)
        s_local = lax.rem(s, STILE)
        b2 = lax.rem(s, 2)

        @pl.when(s_local == 0)
        def _():
            for sl in range(2):
                @pl.when(slot == sl)
                def _():
                    @pl.when(bk < NSTILE)
                    def _():
                        iblk_copy(bk, sl).wait()

                    @pl.when(bk == NSTILE)
                    def _():
                        itail_copy(sl).wait()

        @pl.when(s >= 2)
        def _():
            for b in range(2):
                @pl.when(b2 == b)
                def _():
                    flush_copy(b, s - 2).wait()

        fill_slab(slot, b2, s_local)

        for b in range(2):
            @pl.when(b2 == b)
            def _():
                flush_copy(b, s).start()

        @pl.when(s_local == STILE - 1)
        def _():
            for sl in range(2):
                @pl.when(slot == sl)
                def _():
                    @pl.when(bk + 2 < NSTILE)
                    def _():
                        iblk_copy(bk + 2, sl).start()

                    @pl.when(bk + 2 == NSTILE)
                    def _():
                        itail_copy(sl).start()

    flush_copy(0, SEQ - 2).wait()
    flush_copy(1, SEQ - 1).wait()


def kernel(indices, table):
    B, S, P = indices.shape
    idx_t = jnp.transpose(indices.astype(jnp.int32), (2, 1, 0))
    table_flat = table.astype(jnp.float32).reshape(-1)
    out = _gather_rows(idx_t, table_flat)
    return jnp.transpose(out, (1, 0, 2))


# speedup vs baseline: 3.4595x; 1.0046x over previous
import functools

import jax
import jax.numpy as jnp
from jax import lax
from jax.experimental import pallas as pl
from jax.experimental.pallas import tpu as pltpu
from jax.experimental.pallas import tpu_sc as plsc

NUM_EMB = 103
EMB_DIM = 32
BATCH = 1024
SEQ = 50
MAX_PAD = 20
FEAT = MAX_PAD * EMB_DIM

NUM_CORES = 2
NUM_SUBCORES = 16
NUM_WORKERS = NUM_CORES * NUM_SUBCORES
BCOL = BATCH // NUM_WORKERS
SLAB_ITERS = MAX_PAD * (BCOL // 16)
STILE = 8
NSTILE = 6
STAIL = SEQ - NSTILE * STILE


@functools.partial(
    pl.kernel,
    out_type=jax.ShapeDtypeStruct((SEQ, BATCH, FEAT), jnp.float32),
    mesh=plsc.VectorSubcoreMesh(core_axis_name="c", subcore_axis_name="s"),
    scratch_types=[
        pltpu.VMEM((NUM_EMB * EMB_DIM,), jnp.float32),
        pltpu.VMEM((2, MAX_PAD, STILE, 128), jnp.int32),
        pltpu.VMEM((2 * BCOL, FEAT), jnp.float32),
        pltpu.SemaphoreType.DMA,
        pltpu.SemaphoreType.DMA,
        pltpu.SemaphoreType.DMA,
        pltpu.SemaphoreType.DMA,
    ],
    compiler_params=pltpu.CompilerParams(needs_layout_passes=False),
)
def _gather_rows(idx_hbm, table_hbm, out_hbm, table_v, iring, rows,
                 si0, si1, so0, so1):
    wid = lax.axis_index("s") * NUM_CORES + lax.axis_index("c")
    b0 = wid * BCOL
    bb = (wid // 4) * 128
    bq = (wid % 4) * BCOL
    si = (si0, si1)
    so = (so0, so1)

    def iblk_copy(bk, slot):
        return pltpu.make_async_copy(
            idx_hbm.at[:, pl.ds(bk * STILE, STILE), pl.ds(bb, 128)],
            iring.at[slot], si[slot])

    def itail_copy(slot):
        return pltpu.make_async_copy(
            idx_hbm.at[:, pl.ds(NSTILE * STILE, STAIL), pl.ds(bb, 128)],
            iring.at[slot].at[:, pl.ds(0, STAIL)], si[slot])

    iblk_copy(0, 0).start()
    iblk_copy(1, 1).start()
    pltpu.sync_copy(table_hbm, table_v)

    lane = lax.iota(jnp.int32, 16)
    ones_mask = lane >= 0
    offs = [(lane + c) & (EMB_DIM - 1) for c in range(EMB_DIM)]

    def fill_slab(slot, b2, s_local):
        @plsc.parallel_loop(0, SLAB_ITERS, 1, unroll=2)
        def slab_body(m):
            p = m >> 1
            bg = m & 1
            vidx = plsc.load_gather(
                iring, [jnp.full((16,), slot, jnp.int32),
                        jnp.full((16,), p, jnp.int32),
                        jnp.full((16,), s_local, jnp.int32),
                        lane + (bq + bg * 16)],
                mask=ones_mask)
            vbase = vidx * EMB_DIM
            rowv = lane + (b2 * BCOL + bg * 16)
            p32 = p * EMB_DIM
            for c in range(EMB_DIM):
                col = plsc.load_gather(table_v, [vbase + offs[c]],
                                       mask=ones_mask)
                plsc.store_scatter(rows, [rowv, offs[c] + p32], col,
                                   mask=ones_mask)

    def flush_copy(b2s, s):
        return pltpu.make_async_copy(rows.at[pl.ds(b2s * BCOL, BCOL)],
                                     out_hbm.at[s, pl.ds(b0, BCOL)], so[b2s])

    @pl.loop(0, SEQ)
    def s_loop(s):
        bk = s >> 3
        slot = lax.rem(bk, 2)
        s_local = lax.rem(s, STILE)
        b2 = lax.rem(s, 2)

        @pl.when(s_local == 0)
        def _():
            for sl in range(2):
                @pl.when(slot == sl)
                def _():
                    @pl.when(bk < NSTILE)
                    def _():
                        iblk_copy(bk, sl).wait()

                    @pl.when(bk == NSTILE)
                    def _():
                        itail_copy(sl).wait()

        @pl.when(s >= 2)
        def _():
            for b in range(2):
                @pl.when(b2 == b)
                def _():
                    flush_copy(b, s - 2).wait()

        fill_slab(slot, b2, s_local)

        for b in range(2):
            @pl.when(b2 == b)
            def _():
                flush_copy(b, s).start()

        @pl.when(s_local == STILE - 1)
        def _():
            for sl in range(2):
                @pl.when(slot == sl)
                def _():
                    @pl.when(bk + 2 < NSTILE)
                    def _():
                        iblk_copy(bk + 2, sl).start()

                    @pl.when(bk + 2 == NSTILE)
                    def _():
                        itail_copy(sl).start()

    flush_copy(0, SEQ - 2).wait()
    flush_copy(1, SEQ - 1).wait()


def kernel(indices, table):
    B, S, P = indices.shape
    idx_t = jnp.transpose(indices.astype(jnp.int32), (2, 1, 0))
    table_flat = table.astype(jnp.float32).reshape(-1)
    out = _gather_rows(idx_t, table_flat)
    return jnp.transpose(out, (1, 0, 2))
